# Initial kernel scaffold; baseline (speedup 1.0000x reference)
import jax
import jax.numpy as jnp
from jax.experimental import pallas as pl


def kernel(fpn_masks, out_cls_logits, out_offsets, out_rois, out_scores, out_roimask, cls_log, gt_cls, gt_offsets, gt_segments, segments_label, segments_mask):
    raise NotImplementedError("write your pallas kernel here")


# speedup vs baseline: 6.8898x; 6.8898x over previous
import numpy as np
import jax
import jax.numpy as jnp
from jax.experimental import pallas as pl
from jax.experimental.pallas import tpu as pltpu

_Nr = 1000
_Ng = 32
_C = 200
_B = 8
_T = 4032

_FG_IOU = 0.7
_BG_IOU = 0.01


def _body(rl_ref, rr_ref, gl_ref, gr_ref, lab_ref, sc_ref, cls_ref, tri_ref,
          xl_ref, gcf_ref, fm_ref, opl_ref, opr_ref, gpl_ref, gpr_ref,
          out_ref, acc_ref):
    j = pl.program_id(0)

    @pl.when(j == 0)
    def _init():
        for i in range(5):
            acc_ref[i] = 0.0

    rl = rl_ref[0]
    rr = rr_ref[0]
    gl = gl_ref[0]
    gr = gr_ref[0]
    min_l = jnp.minimum(gl, rl)
    max_l = jnp.maximum(gl, rl)
    min_r = jnp.minimum(gr, rr)
    max_r = jnp.maximum(gr, rr)
    mat = (min_r - max_l) / (max_r - min_l)
    ious = jnp.max(mat, axis=1, keepdims=True)
    kio = jax.lax.broadcasted_iota(jnp.int32, (_Nr, _Ng), 1)
    idx = jnp.min(jnp.where(mat >= ious, kio, _Ng), axis=1, keepdims=True)
    labf = lab_ref[0]
    iou_lab = jnp.sum(jnp.where(kio == idx, labf, 0.0), axis=1, keepdims=True)
    posf = (ious > _FG_IOU).astype(jnp.float32)
    npos = jnp.sum(posf)
    sc = sc_ref[0]
    bgf = jnp.where((ious < _BG_IOU) & (sc > 0.0), 1.0, 0.0)
    cum = jnp.dot(tri_ref[...], bgf, preferred_element_type=jnp.float32)
    bg_sel = bgf * (cum < npos + 0.5).astype(jnp.float32)
    sel = jnp.maximum(posf, bg_sel)
    labels = iou_lab * posf

    cls = cls_ref[0]
    rowmax = jnp.max(cls, axis=1, keepdims=True)
    esum = jnp.sum(jnp.exp(cls - rowmax), axis=1, keepdims=True)
    lse = rowmax + jnp.log(esum)
    cio = jax.lax.broadcasted_iota(jnp.int32, (_Nr, _C), 1)
    labi = labels.astype(jnp.int32)
    picked = jnp.sum(jnp.where(cio == labi, cls, 0.0), axis=1, keepdims=True)
    ce = lse - picked
    acc_ref[0] = acc_ref[0] + jnp.sum(ce * sel)
    acc_ref[1] = acc_ref[1] + jnp.sum(sel)

    x = xl_ref[0]
    g = gcf_ref[0]
    m = fm_ref[0]
    t = (g > 0.5).astype(jnp.float32)
    ax = jnp.abs(x)
    l1p = jnp.log1p(jnp.exp(-ax))
    ls_pos = jnp.minimum(x, 0.0) - l1p
    ls_neg = jnp.minimum(-x, 0.0) - l1p
    ce_f = -(t * ls_pos + (1.0 - t) * ls_neg)
    p = 1.0 / (1.0 + jnp.exp(-x))
    p_t = p * t + (1.0 - p) * (1.0 - t)
    q = 1.0 - p_t
    alpha_t = 0.25 * t + 0.75 * (1.0 - t)
    fl = alpha_t * ce_f * q * q
    acc_ref[2] = acc_ref[2] + jnp.sum(fl * m)
    posm = t * m
    acc_ref[4] = acc_ref[4] + jnp.sum(posm)

    lp = opl_ref[0]
    rp = opr_ref[0]
    lg = gpl_ref[0]
    rg = gpr_ref[0]
    intsctk = jnp.minimum(rp, rg) + jnp.minimum(lp, lg)
    unionk = (lp + rp) + (lg + rg) - intsctk
    iouk = intsctk / jnp.maximum(unionk, 1e-8)
    len_c = jnp.maximum(lp, lg) + jnp.maximum(rp, rg)
    rho = 0.5 * (rp - lp - rg + lg)
    rr_ = rho / jnp.maximum(len_c, 1e-8)
    dl = 1.0 - iouk + rr_ * rr_
    acc_ref[3] = acc_ref[3] + jnp.sum(dl * posm)

    @pl.when(j == _B - 1)
    def _fin():
        norm = 90.0 + 0.1 * jnp.maximum(acc_ref[4], 1.0)
        out_ref[0, 0] = (acc_ref[2] + acc_ref[3]) / norm + acc_ref[0] / acc_ref[1]


_TRI = np.tri(_Nr, dtype=np.float32)


def kernel(fpn_masks, out_cls_logits, out_offsets, out_rois, out_scores,
           out_roimask, cls_log, gt_cls, gt_offsets, gt_segments,
           segments_label, segments_mask):
    f32 = jnp.float32
    rl = out_rois[:, :, 1:2]
    rr = out_rois[:, :, 2:3]
    gl = gt_segments[:, None, :, 0]
    gr = gt_segments[:, None, :, 1]
    lab = segments_label.astype(f32)[:, None, :]
    sc = out_scores[:, :, None]
    tri = jnp.asarray(_TRI)
    t2 = (_B, 8, _T // 8)
    xl = out_cls_logits.reshape(t2)
    gcf = gt_cls.astype(f32).reshape(t2)
    fm = fpn_masks.astype(f32).reshape(t2)
    opl = out_offsets[:, :, 0].reshape(t2)
    opr = out_offsets[:, :, 1].reshape(t2)
    gpl = gt_offsets[:, :, 0].reshape(t2)
    gpr = gt_offsets[:, :, 1].reshape(t2)

    col = pl.BlockSpec((1, _Nr, 1), lambda j: (j, 0, 0))
    row32 = pl.BlockSpec((1, 1, _Ng), lambda j: (j, 0, 0))
    pts = pl.BlockSpec((1, 8, _T // 8), lambda j: (j, 0, 0))

    out = pl.pallas_call(
        _body,
        grid=(_B,),
        in_specs=[
            col, col, row32, row32, row32, col,
            pl.BlockSpec((1, _Nr, _C), lambda j: (j, 0, 0)),
            pl.BlockSpec((_Nr, _Nr), lambda j: (0, 0)),
            pts, pts, pts, pts, pts, pts, pts,
        ],
        out_specs=pl.BlockSpec((1, 1), lambda j: (0, 0), memory_space=pltpu.SMEM),
        out_shape=jax.ShapeDtypeStruct((1, 1), f32),
        scratch_shapes=[pltpu.SMEM((8,), f32)],
    )(rl, rr, gl, gr, lab, sc, cls_log, tri, xl, gcf, fm, opl, opr, gpl, gpr)
    return out[0, 0]


# speedup vs baseline: 7.3958x; 1.0734x over previous
import numpy as np
import jax
import jax.numpy as jnp
from jax.experimental import pallas as pl
from jax.experimental.pallas import tpu as pltpu

_Nr = 1000
_Ng = 32
_C = 200
_B = 8
_T = 4032

_FG_IOU = 0.7
_BG_IOU = 0.01


def _body(rois_ref, gl_ref, gr_ref, lab_ref, sc_ref, cls_ref, tri_ref,
          xl_ref, gci_ref, fm_ref, opl_ref, opr_ref, gpl_ref, gpr_ref,
          out_ref, acc_ref):
    j = pl.program_id(0)

    @pl.when(j == 0)
    def _init():
        for i in range(5):
            acc_ref[i] = 0.0

    rois = rois_ref[0]
    rl = rois[:, 1:2]
    rr = rois[:, 2:3]
    gl = gl_ref[0]
    gr = gr_ref[0]
    min_l = jnp.minimum(gl, rl)
    max_l = jnp.maximum(gl, rl)
    min_r = jnp.minimum(gr, rr)
    max_r = jnp.maximum(gr, rr)
    mat = (min_r - max_l) / (max_r - min_l)
    ious = jnp.max(mat, axis=1, keepdims=True)
    kio = jax.lax.broadcasted_iota(jnp.int32, (_Nr, _Ng), 1)
    idx = jnp.min(jnp.where(mat >= ious, kio, _Ng), axis=1, keepdims=True)
    labf = lab_ref[0]
    iou_lab = jnp.sum(jnp.where(kio == idx, labf, 0.0), axis=1, keepdims=True)
    posf = (ious > _FG_IOU).astype(jnp.float32)
    npos = jnp.sum(posf)
    sc = sc_ref[0]
    bgf = jnp.where((ious < _BG_IOU) & (sc > 0.0), 1.0, 0.0)
    cum = jnp.dot(tri_ref[...], bgf.astype(jnp.bfloat16),
                  preferred_element_type=jnp.float32)
    bg_sel = bgf * (cum < npos + 0.5).astype(jnp.float32)
    sel = jnp.maximum(posf, bg_sel)
    labels = iou_lab * posf

    cls = cls_ref[0]
    rowmax = jnp.max(cls, axis=1, keepdims=True)
    esum = jnp.sum(jnp.exp(cls - rowmax), axis=1, keepdims=True)
    lse = rowmax + jnp.log(esum)
    cio = jax.lax.broadcasted_iota(jnp.int32, (_Nr, _C), 1)
    labi = labels.astype(jnp.int32)
    picked = jnp.sum(jnp.where(cio == labi, cls, 0.0), axis=1, keepdims=True)
    ce = lse - picked
    acc_ref[0] = acc_ref[0] + jnp.sum(ce * sel)
    acc_ref[1] = acc_ref[1] + jnp.sum(sel)

    x = xl_ref[0]
    g = gci_ref[0].astype(jnp.float32)
    m = fm_ref[0].astype(jnp.float32)
    t = (g > 0.5).astype(jnp.float32)
    ax = jnp.abs(x)
    l1p = jnp.log1p(jnp.exp(-ax))
    ls_pos = jnp.minimum(x, 0.0) - l1p
    ls_neg = jnp.minimum(-x, 0.0) - l1p
    ce_f = -(t * ls_pos + (1.0 - t) * ls_neg)
    p = 1.0 / (1.0 + jnp.exp(-x))
    p_t = p * t + (1.0 - p) * (1.0 - t)
    q = 1.0 - p_t
    alpha_t = 0.25 * t + 0.75 * (1.0 - t)
    fl = alpha_t * ce_f * q * q
    acc_ref[2] = acc_ref[2] + jnp.sum(fl * m)
    posm = t * m
    acc_ref[4] = acc_ref[4] + jnp.sum(posm)

    lp = opl_ref[0]
    rp = opr_ref[0]
    lg = gpl_ref[0]
    rg = gpr_ref[0]
    intsctk = jnp.minimum(rp, rg) + jnp.minimum(lp, lg)
    unionk = (lp + rp) + (lg + rg) - intsctk
    iouk = intsctk / jnp.maximum(unionk, 1e-8)
    len_c = jnp.maximum(lp, lg) + jnp.maximum(rp, rg)
    rho = 0.5 * (rp - lp - rg + lg)
    rr_ = rho / jnp.maximum(len_c, 1e-8)
    dl = 1.0 - iouk + rr_ * rr_
    acc_ref[3] = acc_ref[3] + jnp.sum(dl * posm)

    @pl.when(j == _B - 1)
    def _fin():
        norm = 90.0 + 0.1 * jnp.maximum(acc_ref[4], 1.0)
        out_ref[0, 0] = (acc_ref[2] + acc_ref[3]) / norm + acc_ref[0] / acc_ref[1]


_TRI = np.tri(_Nr, dtype=np.float32).astype(jnp.bfloat16)


def kernel(fpn_masks, out_cls_logits, out_offsets, out_rois, out_scores,
           out_roimask, cls_log, gt_cls, gt_offsets, gt_segments,
           segments_label, segments_mask):
    f32 = jnp.float32
    tri = jnp.asarray(_TRI)
    gl = gt_segments[:, None, :, 0]
    gr = gt_segments[:, None, :, 1]
    lab = segments_label.astype(f32)[:, None, :]
    sc = out_scores[:, :, None]
    t2 = (_B, 8, _T // 8)
    xl = out_cls_logits.reshape(t2)
    gci = gt_cls.reshape(t2)
    fm = fpn_masks.reshape(t2)
    opl = out_offsets[:, :, 0].reshape(t2)
    opr = out_offsets[:, :, 1].reshape(t2)
    gpl = gt_offsets[:, :, 0].reshape(t2)
    gpr = gt_offsets[:, :, 1].reshape(t2)

    col = pl.BlockSpec((1, _Nr, 1), lambda j: (j, 0, 0))
    row32 = pl.BlockSpec((1, 1, _Ng), lambda j: (j, 0, 0))
    pts = pl.BlockSpec((1, 8, _T // 8), lambda j: (j, 0, 0))

    out = pl.pallas_call(
        _body,
        grid=(_B,),
        in_specs=[
            pl.BlockSpec((1, _Nr, 3), lambda j: (j, 0, 0)),
            row32, row32, row32, col,
            pl.BlockSpec((1, _Nr, _C), lambda j: (j, 0, 0)),
            pl.BlockSpec((_Nr, _Nr), lambda j: (0, 0)),
            pts, pts, pts, pts, pts, pts, pts,
        ],
        out_specs=pl.BlockSpec((1, 1), lambda j: (0, 0), memory_space=pltpu.SMEM),
        out_shape=jax.ShapeDtypeStruct((1, 1), f32),
        scratch_shapes=[pltpu.SMEM((8,), f32)],
    )(out_rois, gl, gr, lab, sc, cls_log, tri, xl, gci, fm, opl, opr, gpl, gpr)
    return out[0, 0]


# speedup vs baseline: 7.6100x; 1.0290x over previous
import numpy as np
import jax
import jax.numpy as jnp
from jax.experimental import pallas as pl
from jax.experimental.pallas import tpu as pltpu

_Nr = 1000
_Ng = 32
_C = 200
_B = 8
_T = 4032

_FG_IOU = 0.7
_BG_IOU = 0.01


def _body(rois_ref, gl_ref, gr_ref, lab_ref, sc_ref, cls_ref, tri_ref,
          xl_ref, gci_ref, fm_ref, opl_ref, opr_ref, gpl_ref, gpr_ref,
          out_ref, acc_ref):
    j = pl.program_id(0)

    @pl.when(j == 0)
    def _init():
        for i in range(5):
            acc_ref[i] = 0.0

    rois = rois_ref[0]
    rl = rois[:, 1:2]
    rr = rois[:, 2:3]
    gl = gl_ref[0]
    gr = gr_ref[0]
    min_l = jnp.minimum(gl, rl)
    max_l = jnp.maximum(gl, rl)
    min_r = jnp.minimum(gr, rr)
    max_r = jnp.maximum(gr, rr)
    mat = (min_r - max_l) / (max_r - min_l)
    ious = jnp.max(mat, axis=1, keepdims=True)
    labf = lab_ref[0]
    kio = jax.lax.broadcasted_iota(jnp.int32, (1, _Ng), 1)
    keyrow = ((_Ng - kio) * 256).astype(jnp.float32) + labf
    keym = jnp.max(jnp.where(mat >= ious, keyrow, 0.0), axis=1, keepdims=True)
    iou_lab = keym - 256.0 * jnp.floor(keym * (1.0 / 256.0))
    posf = (ious > _FG_IOU).astype(jnp.float32)
    npos = jnp.sum(posf)
    sc = sc_ref[0]
    bgf = jnp.where((ious < _BG_IOU) & (sc > 0.0), 1.0, 0.0)
    cum = jnp.dot(tri_ref[...], bgf.astype(jnp.bfloat16),
                  preferred_element_type=jnp.float32)
    bg_sel = bgf * (cum < npos + 0.5).astype(jnp.float32)
    sel = jnp.maximum(posf, bg_sel)
    labels = iou_lab * posf

    cls = cls_ref[0]
    rowmax = jnp.max(cls, axis=1, keepdims=True)
    esum = jnp.sum(jnp.exp(cls - rowmax), axis=1, keepdims=True)
    lse = rowmax + jnp.log(esum)
    cio = jax.lax.broadcasted_iota(jnp.int32, (_Nr, _C), 1)
    labi = labels.astype(jnp.int32)
    picked = jnp.sum(jnp.where(cio == labi, cls, 0.0), axis=1, keepdims=True)
    ce = lse - picked
    acc_ref[0] = acc_ref[0] + jnp.sum(ce * sel)
    acc_ref[1] = acc_ref[1] + jnp.sum(sel)

    x = xl_ref[0]
    g = gci_ref[0].astype(jnp.float32)
    m = fm_ref[0].astype(jnp.float32)
    t = (g > 0.5).astype(jnp.float32)
    ax = jnp.abs(x)
    l1p = jnp.log1p(jnp.exp(-ax))
    ls_pos = jnp.minimum(x, 0.0) - l1p
    ls_neg = jnp.minimum(-x, 0.0) - l1p
    ce_f = -(t * ls_pos + (1.0 - t) * ls_neg)
    p = 1.0 / (1.0 + jnp.exp(-x))
    p_t = p * t + (1.0 - p) * (1.0 - t)
    q = 1.0 - p_t
    alpha_t = 0.25 * t + 0.75 * (1.0 - t)
    fl = alpha_t * ce_f * q * q
    acc_ref[2] = acc_ref[2] + jnp.sum(fl * m)
    posm = t * m
    acc_ref[4] = acc_ref[4] + jnp.sum(posm)

    lp = opl_ref[0]
    rp = opr_ref[0]
    lg = gpl_ref[0]
    rg = gpr_ref[0]
    intsctk = jnp.minimum(rp, rg) + jnp.minimum(lp, lg)
    unionk = (lp + rp) + (lg + rg) - intsctk
    iouk = intsctk / jnp.maximum(unionk, 1e-8)
    len_c = jnp.maximum(lp, lg) + jnp.maximum(rp, rg)
    rho = 0.5 * (rp - lp - rg + lg)
    rr_ = rho / jnp.maximum(len_c, 1e-8)
    dl = 1.0 - iouk + rr_ * rr_
    acc_ref[3] = acc_ref[3] + jnp.sum(dl * posm)

    @pl.when(j == _B - 1)
    def _fin():
        norm = 90.0 + 0.1 * jnp.maximum(acc_ref[4], 1.0)
        out_ref[0, 0] = (acc_ref[2] + acc_ref[3]) / norm + acc_ref[0] / acc_ref[1]


_TRI = np.tri(_Nr, dtype=np.float32).astype(jnp.bfloat16)


def kernel(fpn_masks, out_cls_logits, out_offsets, out_rois, out_scores,
           out_roimask, cls_log, gt_cls, gt_offsets, gt_segments,
           segments_label, segments_mask):
    f32 = jnp.float32
    tri = jnp.asarray(_TRI)
    gl = gt_segments[:, None, :, 0]
    gr = gt_segments[:, None, :, 1]
    lab = segments_label.astype(f32)[:, None, :]
    sc = out_scores[:, :, None]
    t2 = (_B, 8, _T // 8)
    xl = out_cls_logits.reshape(t2)
    gci = gt_cls.reshape(t2)
    fm = fpn_masks.reshape(t2)
    opl = out_offsets[:, :, 0].reshape(t2)
    opr = out_offsets[:, :, 1].reshape(t2)
    gpl = gt_offsets[:, :, 0].reshape(t2)
    gpr = gt_offsets[:, :, 1].reshape(t2)

    col = pl.BlockSpec((1, _Nr, 1), lambda j: (j, 0, 0))
    row32 = pl.BlockSpec((1, 1, _Ng), lambda j: (j, 0, 0))
    pts = pl.BlockSpec((1, 8, _T // 8), lambda j: (j, 0, 0))

    out = pl.pallas_call(
        _body,
        grid=(_B,),
        in_specs=[
            pl.BlockSpec((1, _Nr, 3), lambda j: (j, 0, 0)),
            row32, row32, row32, col,
            pl.BlockSpec((1, _Nr, _C), lambda j: (j, 0, 0)),
            pl.BlockSpec((_Nr, _Nr), lambda j: (0, 0)),
            pts, pts, pts, pts, pts, pts, pts,
        ],
        out_specs=pl.BlockSpec((1, 1), lambda j: (0, 0), memory_space=pltpu.SMEM),
        out_shape=jax.ShapeDtypeStruct((1, 1), f32),
        scratch_shapes=[pltpu.SMEM((8,), f32)],
    )(out_rois, gl, gr, lab, sc, cls_log, tri, xl, gci, fm, opl, opr, gpl, gpr)
    return out[0, 0]
